# speedup vs baseline: 1.0121x; 1.0121x over previous
import jax
import jax.numpy as jnp
from jax.experimental import pallas as pl

N = 10000
HEADS = 8
HID = 128
OUT = 128


def _matmul_kernel(x_ref, w_ref, o_ref):
    o_ref[...] = jnp.dot(x_ref[...], w_ref[...],
                         preferred_element_type=jnp.float32)


def _matmul(x, w):
    n, k = x.shape
    _, m = w.shape
    bn = 1024
    n_pad = ((n + bn - 1) // bn) * bn
    x = jnp.pad(x, ((0, n_pad - n), (0, 0)))
    out = pl.pallas_call(
        _matmul_kernel,
        grid=(n_pad // bn,),
        in_specs=[pl.BlockSpec((bn, k), lambda i: (i, 0)),
                  pl.BlockSpec((k, m), lambda i: (0, 0))],
        out_specs=pl.BlockSpec((bn, m), lambda i: (i, 0)),
        out_shape=jax.ShapeDtypeStruct((n_pad, m), jnp.float32),
    )(x, w)
    return out[:n]


def _gat_conv(x, src, dst, W, a_src, a_dst, b, heads, ch):
    n = x.shape[0]
    h = _matmul(x, W).reshape(n, heads, ch)
    alpha_src = jnp.sum(h * a_src[None, :, :], axis=-1)
    alpha_dst = jnp.sum(h * a_dst[None, :, :], axis=-1)
    e = alpha_src[src] + alpha_dst[dst]
    e = jax.nn.leaky_relu(e, negative_slope=0.2)
    m = jax.ops.segment_max(e, dst, num_segments=n)
    e = jnp.exp(e - m[dst])
    denom = jax.ops.segment_sum(e, dst, num_segments=n)
    alpha = e / (denom[dst] + 1e-16)
    msg = h[src] * alpha[:, :, None]
    out = jax.ops.segment_sum(msg, dst, num_segments=n)
    return out.reshape(n, heads * ch) + b


def kernel(x, edge_index, W1, a_src1, a_dst1, b1, W2, a_src2, a_dst2, b2):
    n = x.shape[0]
    loop = jnp.arange(n, dtype=edge_index.dtype)
    src = jnp.concatenate([edge_index[0], loop])
    dst = jnp.concatenate([edge_index[1], loop])
    h = _gat_conv(x, src, dst, W1, a_src1, a_dst1, b1, HEADS, HID)
    h = jax.nn.relu(h)
    out = _gat_conv(h, src, dst, W2, a_src2, a_dst2, b2, 1, OUT)
    return out


# speedup vs baseline: 10.9518x; 10.8212x over previous
import functools
import jax
import jax.numpy as jnp
from jax import lax
from jax.experimental import pallas as pl
from jax.experimental.pallas import tpu as pltpu
from jax.experimental.pallas import tpu_sc as plsc

N = 10000
NP = 10016
HEADS = 8
HID = 128
OUT = 128
E_RAW = 320000
E1 = E_RAW + N
NSC = 2
NTILE = 16
CE_A = 10368
E2 = 32 * CE_A
BA = 864
CE_B = E2 // NTILE
BB = 128
ND8 = NP * 8
NROW = NP // NTILE
NPD = NTILE * 640


def _mesh():
    return plsc.VectorSubcoreMesh(core_axis_name="c", subcore_axis_name="s")


def _tc1_body(x_ref, w1_ref, a1s_ref, a1d_ref, h1_ref, as_ref, ad_ref):
    hb = jnp.dot(x_ref[...], w1_ref[...], preferred_element_type=jnp.float32)
    bn = hb.shape[0]
    h3 = hb.reshape(bn, HEADS, HID)
    as_ref[...] = jnp.sum(h3 * a1s_ref[...][None], axis=-1)
    ad_ref[...] = jnp.sum(h3 * a1d_ref[...][None], axis=-1)
    h1_ref[...] = h3.transpose(1, 0, 2)


def _tc1(x_pad, W1, a_src1, a_dst1):
    bn = 2504
    grid = NP // bn
    return pl.pallas_call(
        _tc1_body,
        grid=(grid,),
        in_specs=[
            pl.BlockSpec((bn, 128), lambda i: (i, 0)),
            pl.BlockSpec((128, HEADS * HID), lambda i: (0, 0)),
            pl.BlockSpec((HEADS, HID), lambda i: (0, 0)),
            pl.BlockSpec((HEADS, HID), lambda i: (0, 0)),
        ],
        out_specs=[
            pl.BlockSpec((HEADS, bn, HID), lambda i: (0, i, 0)),
            pl.BlockSpec((bn, HEADS), lambda i: (i, 0)),
            pl.BlockSpec((bn, HEADS), lambda i: (i, 0)),
        ],
        out_shape=[
            jax.ShapeDtypeStruct((HEADS, NP, HID), jnp.float32),
            jax.ShapeDtypeStruct((NP, HEADS), jnp.float32),
            jax.ShapeDtypeStruct((NP, HEADS), jnp.float32),
        ],
    )(x_pad, W1, a_src1, a_dst1)


def _tc2_body(o1_ref, den_ref, b1_ref, w2_ref, a2s_ref, a2d_ref,
              h2_ref, as_ref, ad_ref):
    bn = o1_ref.shape[1]
    dsum = jnp.sum(den_ref[...].reshape(bn, NSC * NTILE, HEADS),
                   axis=1) + 1e-16
    acc = jnp.zeros((bn, OUT), jnp.float32)
    for h in range(HEADS):
        g = o1_ref[h] / dsum[:, h:h + 1] + b1_ref[h][None]
        g = jnp.maximum(g, 0.0)
        acc = acc + jnp.dot(g, w2_ref[h], preferred_element_type=jnp.float32)
    h2_ref[...] = acc
    as_ref[...] = jnp.sum(acc * a2s_ref[...], axis=-1, keepdims=True)
    ad_ref[...] = jnp.sum(acc * a2d_ref[...], axis=-1, keepdims=True)


def _tc2(out1, den1, b1r, W2r, a_src2, a_dst2):
    bn = 2504
    grid = NP // bn
    return pl.pallas_call(
        _tc2_body,
        grid=(grid,),
        in_specs=[
            pl.BlockSpec((HEADS, bn, HID), lambda i: (0, i, 0)),
            pl.BlockSpec((bn, NSC * NTILE * HEADS), lambda i: (i, 0)),
            pl.BlockSpec((HEADS, HID), lambda i: (0, 0)),
            pl.BlockSpec((HEADS, HID, OUT), lambda i: (0, 0, 0)),
            pl.BlockSpec((1, OUT), lambda i: (0, 0)),
            pl.BlockSpec((1, OUT), lambda i: (0, 0)),
        ],
        out_specs=[
            pl.BlockSpec((bn, OUT), lambda i: (i, 0)),
            pl.BlockSpec((bn, 1), lambda i: (i, 0)),
            pl.BlockSpec((bn, 1), lambda i: (i, 0)),
        ],
        out_shape=[
            jax.ShapeDtypeStruct((NP, OUT), jnp.float32),
            jax.ShapeDtypeStruct((NP, 1), jnp.float32),
            jax.ShapeDtypeStruct((NP, 1), jnp.float32),
        ],
    )(out1, den1, b1r, W2r, a_src2, a_dst2)


def _tc3_body(o2_ref, d2_ref, b2_ref, out_ref):
    d = jnp.sum(d2_ref[...], axis=1, keepdims=True) + 1e-16
    out_ref[...] = (o2_ref[0] + o2_ref[1]) / d + b2_ref[...]


def _tc3(out2, den2, b2):
    bn = 2504
    grid = NP // bn
    return pl.pallas_call(
        _tc3_body,
        grid=(grid,),
        in_specs=[
            pl.BlockSpec((NSC, bn, OUT), lambda i: (0, i, 0)),
            pl.BlockSpec((bn, NSC), lambda i: (i, 0)),
            pl.BlockSpec((1, OUT), lambda i: (0, 0)),
        ],
        out_specs=pl.BlockSpec((bn, OUT), lambda i: (i, 0)),
        out_shape=jax.ShapeDtypeStruct((NP, OUT), jnp.float32),
    )(out2, den2, b2.reshape(1, OUT))


def _sca_body(src_hbm, dst_hbm, asrc_hbm, adst_hbm,
              w_hbm, den_hbm,
              sidx, didx, sidx8, didx8, av, bv, wbuf, den, sem):
    c = lax.axis_index("c")
    s = lax.axis_index("s")
    wid = c * NTILE + s
    ebase = wid * CE_A

    def zbody(i, _):
        den[pl.ds(i * 16, 16)] = jnp.zeros((16,), jnp.float32)
        return 0
    lax.fori_loop(0, ND8 // 16, zbody, 0)

    lane = lax.iota(jnp.int32, 16)
    half = lane >> 3
    lanelow = lane & 7

    def blk_body(blk, _):
        eb = ebase + blk * BA
        pltpu.sync_copy(src_hbm.at[pl.ds(eb, BA)], sidx)
        pltpu.sync_copy(dst_hbm.at[pl.ds(eb, BA)], didx)

        def ibody(i, _):
            e2 = i * 2 + half
            sv = plsc.load_gather(sidx, [e2])
            dv = plsc.load_gather(didx, [e2])
            sidx8[pl.ds(i * 16, 16)] = sv * 8 + lanelow
            didx8[pl.ds(i * 16, 16)] = dv * 8 + lanelow
            return 0
        lax.fori_loop(0, (BA * 8) // 16, ibody, 0)

        pltpu.async_copy(asrc_hbm.at[sidx8], av, sem).wait()
        pltpu.async_copy(adst_hbm.at[didx8], bv, sem).wait()

        def gbody(i, _):
            v = av[pl.ds(i * 16, 16)] + bv[pl.ds(i * 16, 16)]
            v = jnp.maximum(v, 0.2 * v)
            v = jnp.exp(v)
            wbuf[pl.ds(i * 16, 16)] = v
            d16 = didx8[pl.ds(i * 16, 16)]
            plsc.addupdate_scatter(den, [d16], v, mask=half == 0)
            plsc.addupdate_scatter(den, [d16], v, mask=half == 1)
            return 0
        lax.fori_loop(0, (BA * 8) // 16, gbody, 0)

        pltpu.sync_copy(wbuf, w_hbm.at[pl.ds(eb * 8, BA * 8)])
        return 0
    lax.fori_loop(0, CE_A // BA, blk_body, 0)

    pltpu.sync_copy(den, den_hbm.at[pl.ds(wid * ND8, ND8)])


def _sc_a(src, dst, asrc, adst):
    kern = pl.kernel(
        _sca_body,
        mesh=_mesh(),
        compiler_params=pltpu.CompilerParams(needs_layout_passes=False),
        out_type=[
            jax.ShapeDtypeStruct((E2 * 8,), jnp.float32),
            jax.ShapeDtypeStruct((NSC * NTILE * ND8,), jnp.float32),
        ],
        scratch_types=[
            pltpu.VMEM((BA,), jnp.int32),
            pltpu.VMEM((BA,), jnp.int32),
            pltpu.VMEM((BA * 8,), jnp.int32),
            pltpu.VMEM((BA * 8,), jnp.int32),
            pltpu.VMEM((BA * 8,), jnp.float32),
            pltpu.VMEM((BA * 8,), jnp.float32),
            pltpu.VMEM((BA * 8,), jnp.float32),
            pltpu.VMEM((ND8,), jnp.float32),
            pltpu.SemaphoreType.DMA,
        ],
    )
    return kern(src, dst, asrc, adst)


def _scb_body(src_hbm, dst_hbm, w_hbm, h1_hbm, out_hbm,
              sidx, didx, gidx, wv, rows, acc, sem):
    c = lax.axis_index("c")
    s = lax.axis_index("s")
    ebase = s * CE_B
    lane = lax.iota(jnp.int32, 16)

    for p in range(HEADS // NSC):
        h = c * (HEADS // NSC) + p

        def zbody(i, _):
            for jj in range(HID // 16):
                rows[i, pl.ds(jj * 16, 16)] = jnp.zeros((16,), jnp.float32)
            return 0
        lax.fori_loop(0, BB, zbody, 0)
        base = s * 632

        @pl.when(s < NTILE - 1)
        def _zfull():
            for off, nr in ((0, 128), (128, 128), (256, 128),
                            (384, 128), (512, 120)):
                pltpu.sync_copy(rows.at[pl.ds(0, nr)],
                                acc.at[pl.ds(base + off, nr)])

        @pl.when(s == NTILE - 1)
        def _zlast():
            for off, nr in ((0, 128), (128, 128), (256, 128),
                            (384, 128), (512, 24)):
                pltpu.sync_copy(rows.at[pl.ds(0, nr)],
                                acc.at[pl.ds(base + off, nr)])
        plsc.subcore_barrier()

        def blk_body(g, _):
            eb = ebase + g * BB
            pltpu.sync_copy(src_hbm.at[pl.ds(eb, BB)], sidx)
            pltpu.sync_copy(dst_hbm.at[pl.ds(eb, BB)], didx)

            def ibody(j, _):
                v = sidx[pl.ds(j * 16, 16)]
                gidx[pl.ds(j * 16, 16)] = v + h * NP
                return 0
            lax.fori_loop(0, BB // 16, ibody, 0)

            pltpu.sync_copy(w_hbm.at[pl.ds(eb * 8, BB * 8)], wv)
            pltpu.async_copy(h1_hbm.at[gidx], rows, sem).wait()

            def ebody(i, _):
                wb = plsc.load_gather(wv, [jnp.full((16,), 0, jnp.int32) + (i * 8 + h)])
                for jj in range(HID // 16):
                    rows[i, pl.ds(jj * 16, 16)] = (
                        rows[i, pl.ds(jj * 16, 16)] * wb)
                return 0
            lax.fori_loop(0, BB, ebody, 0)

            pltpu.sync_copy(rows, acc.at[didx], add=True)
            return 0
        lax.fori_loop(0, CE_B // BB, blk_body, 0)

        plsc.subcore_barrier()

        @pl.when(s < NTILE - 1)
        def _ffull():
            pltpu.sync_copy(acc.at[pl.ds(base, 632)],
                            out_hbm.at[pl.ds(h * NP + base, 632)])

        @pl.when(s == NTILE - 1)
        def _flast():
            pltpu.sync_copy(acc.at[pl.ds(base, 536)],
                            out_hbm.at[pl.ds(h * NP + base, 536)])
        plsc.subcore_barrier()


def _sc_b(src, dst, w, h1):
    kern = pl.kernel(
        _scb_body,
        mesh=_mesh(),
        compiler_params=pltpu.CompilerParams(needs_layout_passes=False),
        out_type=jax.ShapeDtypeStruct((HEADS * NP, HID), jnp.float32),
        scratch_types=[
            pltpu.VMEM((BB,), jnp.int32),
            pltpu.VMEM((BB,), jnp.int32),
            pltpu.VMEM((BB,), jnp.int32),
            pltpu.VMEM((BB * 8,), jnp.float32),
            pltpu.VMEM((BB, HID), jnp.float32),
            pltpu.VMEM_SHARED((NP, HID), jnp.float32),
            pltpu.SemaphoreType.DMA,
        ],
    )
    return kern(src, dst, w, h1)


def _scc_body(src_hbm, dst_hbm, as_hbm, ad_hbm, h2_hbm,
              out_hbm, den_hbm,
              av, bv, zbuf, sidx, didx, wv, rows, acc, dsh, sem):
    c = lax.axis_index("c")
    s = lax.axis_index("s")
    ebase = (c * NTILE + s) * CE_A

    def zrow(i, _):
        for jj in range(OUT // 16):
            rows[i, pl.ds(jj * 16, 16)] = jnp.zeros((16,), jnp.float32)
        return 0
    lax.fori_loop(0, BB, zrow, 0)

    def zb(i, _):
        zbuf[pl.ds(i * 16, 16)] = jnp.zeros((16,), jnp.float32)
        return 0
    lax.fori_loop(0, 640 // 16, zb, 0)
    base = s * 632

    @pl.when(s < NTILE - 1)
    def _zfull():
        for off, nr in ((0, 128), (128, 128), (256, 128),
                        (384, 128), (512, 120)):
            pltpu.sync_copy(rows.at[pl.ds(0, nr)],
                            acc.at[pl.ds(base + off, nr)])

    @pl.when(s == NTILE - 1)
    def _zlast():
        for off, nr in ((0, 128), (128, 128), (256, 128),
                        (384, 128), (512, 24)):
            pltpu.sync_copy(rows.at[pl.ds(0, nr)],
                            acc.at[pl.ds(base + off, nr)])
    pltpu.sync_copy(zbuf.at[pl.ds(0, 640)], dsh.at[pl.ds(s * 640, 640)])
    plsc.subcore_barrier()

    def blk_body(g, _):
        eb = ebase + g * BB
        pltpu.sync_copy(src_hbm.at[pl.ds(eb, BB)], sidx)
        pltpu.sync_copy(dst_hbm.at[pl.ds(eb, BB)], didx)

        pltpu.async_copy(as_hbm.at[sidx], av, sem).wait()
        pltpu.async_copy(ad_hbm.at[didx], bv, sem).wait()

        def wbody(j, _):
            v = av[pl.ds(j * 16, 16)] + bv[pl.ds(j * 16, 16)]
            v = jnp.maximum(v, 0.2 * v)
            v = jnp.exp(v)
            wv[pl.ds(j * 16, 16)] = v
            return 0
        lax.fori_loop(0, BB // 16, wbody, 0)

        pltpu.sync_copy(wv, dsh.at[didx], add=True)

        pltpu.async_copy(h2_hbm.at[sidx], rows, sem).wait()

        def ebody(i, _):
            wb = plsc.load_gather(wv, [jnp.full((16,), i, jnp.int32)])
            for jj in range(OUT // 16):
                rows[i, pl.ds(jj * 16, 16)] = (
                    rows[i, pl.ds(jj * 16, 16)] * wb)
            return 0
        lax.fori_loop(0, BB, ebody, 0)

        pltpu.sync_copy(rows, acc.at[didx], add=True)
        return 0
    lax.fori_loop(0, CE_A // BB, blk_body, 0)

    plsc.subcore_barrier()

    @pl.when(s < NTILE - 1)
    def _ffull():
        pltpu.sync_copy(acc.at[pl.ds(base, 632)],
                        out_hbm.at[c, pl.ds(base, 632)])

    @pl.when(s == NTILE - 1)
    def _flast():
        pltpu.sync_copy(acc.at[pl.ds(base, 536)],
                        out_hbm.at[c, pl.ds(base, 536)])
    pltpu.sync_copy(dsh.at[pl.ds(s * 640, 640)],
                    den_hbm.at[pl.ds(c * NPD + s * 640, 640)])


def _sc_c(src, dst, asrc2, adst2, h2):
    kern = pl.kernel(
        _scc_body,
        mesh=_mesh(),
        compiler_params=pltpu.CompilerParams(needs_layout_passes=False),
        out_type=[
            jax.ShapeDtypeStruct((NSC, NP, OUT), jnp.float32),
            jax.ShapeDtypeStruct((NSC * NPD,), jnp.float32),
        ],
        scratch_types=[
            pltpu.VMEM((BB,), jnp.float32),
            pltpu.VMEM((BB,), jnp.float32),
            pltpu.VMEM((640,), jnp.float32),
            pltpu.VMEM((BB,), jnp.int32),
            pltpu.VMEM((BB,), jnp.int32),
            pltpu.VMEM((BB,), jnp.float32),
            pltpu.VMEM((BB, OUT), jnp.float32),
            pltpu.VMEM_SHARED((NP, OUT), jnp.float32),
            pltpu.VMEM_SHARED((NPD,), jnp.float32),
            pltpu.SemaphoreType.DMA,
        ],
    )
    return kern(src, dst, asrc2, adst2, h2)


def kernel(x, edge_index, W1, a_src1, a_dst1, b1, W2, a_src2, a_dst2, b2):
    loop = jnp.arange(N, dtype=edge_index.dtype)
    src = jnp.concatenate([edge_index[0], loop]).astype(jnp.int32)
    dst = jnp.concatenate([edge_index[1], loop]).astype(jnp.int32)
    pad = E2 - E1
    src = jnp.concatenate([src, jnp.zeros((pad,), jnp.int32)])
    dst = jnp.concatenate([dst, jnp.full((pad,), N, jnp.int32)])
    x_pad = jnp.pad(x, ((0, NP - N), (0, 0)))

    h1, asrc1, adst1 = _tc1(x_pad, W1, a_src1, a_dst1)
    w1, den1 = _sc_a(src, dst, asrc1.reshape(NP * HEADS),
                     adst1.reshape(NP * HEADS))
    out1 = _sc_b(src, dst, w1, h1.reshape(HEADS * NP, HID))
    den1_t = den1.reshape(NSC * NTILE, NP, HEADS).transpose(1, 0, 2)
    h2, asrc2, adst2 = _tc2(out1.reshape(HEADS, NP, HID),
                            den1_t.reshape(NP, NSC * NTILE * HEADS),
                            b1.reshape(HEADS, HID),
                            W2.reshape(HEADS, HID, OUT), a_src2, a_dst2)
    out2, den2 = _sc_c(src, dst, asrc2.reshape(NP), adst2.reshape(NP), h2)
    out = _tc3(out2, den2.reshape(NSC, NPD)[:, :NP].transpose(1, 0), b2)
    return out[:N]
